# baseline (speedup 1.0000x reference)
import functools

import jax
import jax.numpy as jnp
from jax import lax
from jax.experimental import pallas as pl
from jax.experimental.pallas import tpu as pltpu

NE = 8


def _prologue_body(w_ref, b_ref, x_ref, p_ref, bsum_ref, wsum_ref):
    g = pl.program_id(0)
    f = w_ref.shape[1]
    i_in = f // NE

    @pl.when(g == 0)
    def _():
        row = lax.broadcasted_iota(jnp.int32, (f, i_in), 0)
        col = lax.broadcasted_iota(jnp.int32, (f, i_in), 1)
        fold = (row % i_in == col).astype(jnp.float32)
        parts = []
        for e in range(NE):
            cs = jnp.sum(w_ref[e * i_in:(e + 1) * i_in, :], axis=0,
                         keepdims=True)
            parts.append(lax.dot_general(
                cs, fold, (((1,), (0,)), ((), ())),
                precision=lax.Precision.HIGHEST,
                preferred_element_type=jnp.float32))
        wsum_ref[...] = jnp.concatenate(parts, axis=1)
        bsum_ref[...] = jnp.sum(b_ref[...], keepdims=True).reshape(1, 1)

    xw = x_ref[...] * wsum_ref[...]
    seg_r = lax.broadcasted_iota(jnp.int32, (f, NE), 0)
    seg_c = lax.broadcasted_iota(jnp.int32, (f, NE), 1)
    seg = (seg_r // i_in == seg_c).astype(jnp.float32)
    p_ref[...] = lax.dot_general(xw, seg, (((1,), (0,)), ((), ())),
                                 precision=lax.Precision.HIGHEST,
                                 preferred_element_type=jnp.float32)


def _stream_body(p_ref, d_ref, c_ref, bsum_ref, o_ref, s_ref):
    ph = pl.program_id(1)
    t = pl.program_id(2)
    cap = s_ref.shape[1]

    @pl.when(ph == 0)
    def _():
        @pl.when(t == 0)
        def _():
            s_ref[...] = jnp.broadcast_to(bsum_ref[...], (NE, cap))

        p = p_ref[...]
        incs = [
            jnp.sum(d_ref[:, e, :] * p[:, e:e + 1], axis=0, keepdims=True)
            for e in range(NE)
        ]
        s_ref[...] += jnp.concatenate(incs, axis=0)

    @pl.when(ph == 1)
    def _():
        acc = c_ref[:, 0, :] * s_ref[0:1, :]
        for e in range(1, NE):
            acc += c_ref[:, e, :] * s_ref[e:e + 1, :]
        o_ref[...] = jnp.sum(acc, axis=1, keepdims=True)


def kernel(x, combine_array, dispatch_mask, weight, bias):
    b, t, f = x.shape
    e, c = dispatch_mask.shape[2], dispatch_mask.shape[3]
    assert e == NE
    i_in = f // e
    tblk = 512
    bt = b * t
    nt = t // tblk
    nbt = bt // tblk

    x2 = x.reshape(bt, f)
    d3 = dispatch_mask.reshape(bt, e, c)
    c3 = combine_array.reshape(bt, e, c)

    p, bsum, _ = pl.pallas_call(
        _prologue_body,
        grid=(nbt,),
        in_specs=[
            pl.BlockSpec((f, f), lambda g: (0, 0)),
            pl.BlockSpec((1, f), lambda g: (0, 0)),
            pl.BlockSpec((tblk, f), lambda g: (g, 0)),
        ],
        out_specs=[
            pl.BlockSpec((tblk, NE), lambda g: (g, 0)),
            pl.BlockSpec((1, 1), lambda g: (0, 0)),
            pl.BlockSpec((1, f), lambda g: (0, 0)),
        ],
        out_shape=[
            jax.ShapeDtypeStruct((bt, NE), jnp.float32),
            jax.ShapeDtypeStruct((1, 1), jnp.float32),
            jax.ShapeDtypeStruct((1, f), jnp.float32),
        ],
    )(weight, bias.reshape(1, f), x2)

    out = pl.pallas_call(
        _stream_body,
        grid=(b, 2, nt),
        in_specs=[
            pl.BlockSpec(
                (tblk, NE),
                lambda gb, ph, gt: (gb * nt + jnp.where(ph == 0, gt, nt - 1), 0)),
            pl.BlockSpec(
                (tblk, e, c),
                lambda gb, ph, gt: (gb * nt + jnp.where(ph == 0, gt, nt - 1), 0, 0)),
            pl.BlockSpec(
                (tblk, e, c),
                lambda gb, ph, gt: (gb * nt + jnp.where(ph == 1, gt, 0), 0, 0)),
            pl.BlockSpec((1, 1), lambda gb, ph, gt: (0, 0)),
        ],
        out_specs=pl.BlockSpec(
            (tblk, 1),
            lambda gb, ph, gt: (gb * nt + jnp.where(ph == 1, gt, 0), 0)),
        out_shape=jax.ShapeDtypeStruct((bt, 1), jnp.float32),
        scratch_shapes=[pltpu.VMEM((NE, c), jnp.float32)],
    )(p, d3, c3, bsum)

    return out.reshape(b, t)


# speedup vs baseline: 1.0022x; 1.0022x over previous
import functools

import jax
import jax.numpy as jnp
from jax import lax
from jax.experimental import pallas as pl
from jax.experimental.pallas import tpu as pltpu

NE = 8


def _prologue_body(w_ref, b_ref, x_ref, p_ref, bsum_ref, wsum_ref):
    g = pl.program_id(0)
    f = w_ref.shape[1]
    i_in = f // NE

    @pl.when(g == 0)
    def _():
        row = lax.broadcasted_iota(jnp.int32, (f, i_in), 0)
        col = lax.broadcasted_iota(jnp.int32, (f, i_in), 1)
        fold = (row % i_in == col).astype(jnp.float32)
        parts = []
        for e in range(NE):
            cs = jnp.sum(w_ref[e * i_in:(e + 1) * i_in, :], axis=0,
                         keepdims=True)
            parts.append(lax.dot_general(
                cs, fold, (((1,), (0,)), ((), ())),
                precision=lax.Precision.HIGHEST,
                preferred_element_type=jnp.float32))
        wsum_ref[...] = jnp.concatenate(parts, axis=1)
        bsum_ref[...] = jnp.sum(b_ref[...], keepdims=True).reshape(1, 1)

    xw = x_ref[...] * wsum_ref[...]
    seg_r = lax.broadcasted_iota(jnp.int32, (f, NE), 0)
    seg_c = lax.broadcasted_iota(jnp.int32, (f, NE), 1)
    seg = (seg_r // i_in == seg_c).astype(jnp.float32)
    p_ref[...] = lax.dot_general(xw, seg, (((1,), (0,)), ((), ())),
                                 precision=lax.Precision.HIGHEST,
                                 preferred_element_type=jnp.float32)


def _stream_body(p_ref, d_ref, c_ref, bsum_ref, o_ref, s_ref):
    ph = pl.program_id(1)
    t = pl.program_id(2)
    cap = s_ref.shape[1]

    @pl.when(ph == 0)
    def _():
        @pl.when(t == 0)
        def _():
            s_ref[...] = jnp.broadcast_to(bsum_ref[...], (NE, cap))

        p = p_ref[...]
        incs = [
            lax.dot_general(p[:, e:e + 1], d_ref[:, e, :],
                            (((0,), (0,)), ((), ())),
                            precision=lax.Precision.DEFAULT,
                            preferred_element_type=jnp.float32)
            for e in range(NE)
        ]
        s_ref[...] += jnp.concatenate(incs, axis=0)

    @pl.when(ph == 1)
    def _():
        acc = lax.dot_general(c_ref[:, 0, :], s_ref[0:1, :],
                              (((1,), (1,)), ((), ())),
                              precision=lax.Precision.DEFAULT,
                              preferred_element_type=jnp.float32)
        for e in range(1, NE):
            acc += lax.dot_general(c_ref[:, e, :], s_ref[e:e + 1, :],
                                   (((1,), (1,)), ((), ())),
                                   precision=lax.Precision.DEFAULT,
                                   preferred_element_type=jnp.float32)
        o_ref[...] = acc


def kernel(x, combine_array, dispatch_mask, weight, bias):
    b, t, f = x.shape
    e, c = dispatch_mask.shape[2], dispatch_mask.shape[3]
    assert e == NE
    i_in = f // e
    tblk = 512
    bt = b * t
    nt = t // tblk
    nbt = bt // tblk

    x2 = x.reshape(bt, f)
    d3 = dispatch_mask.reshape(bt, e, c)
    c3 = combine_array.reshape(bt, e, c)

    p, bsum, _ = pl.pallas_call(
        _prologue_body,
        grid=(nbt,),
        in_specs=[
            pl.BlockSpec((f, f), lambda g: (0, 0)),
            pl.BlockSpec((1, f), lambda g: (0, 0)),
            pl.BlockSpec((tblk, f), lambda g: (g, 0)),
        ],
        out_specs=[
            pl.BlockSpec((tblk, NE), lambda g: (g, 0)),
            pl.BlockSpec((1, 1), lambda g: (0, 0)),
            pl.BlockSpec((1, f), lambda g: (0, 0)),
        ],
        out_shape=[
            jax.ShapeDtypeStruct((bt, NE), jnp.float32),
            jax.ShapeDtypeStruct((1, 1), jnp.float32),
            jax.ShapeDtypeStruct((1, f), jnp.float32),
        ],
    )(weight, bias.reshape(1, f), x2)

    out = pl.pallas_call(
        _stream_body,
        grid=(b, 2, nt),
        in_specs=[
            pl.BlockSpec(
                (tblk, NE),
                lambda gb, ph, gt: (gb * nt + jnp.where(ph == 0, gt, nt - 1), 0)),
            pl.BlockSpec(
                (tblk, e, c),
                lambda gb, ph, gt: (gb * nt + jnp.where(ph == 0, gt, nt - 1), 0, 0)),
            pl.BlockSpec(
                (tblk, e, c),
                lambda gb, ph, gt: (gb * nt + jnp.where(ph == 1, gt, 0), 0, 0)),
            pl.BlockSpec((1, 1), lambda gb, ph, gt: (0, 0)),
        ],
        out_specs=pl.BlockSpec(
            (tblk, 1),
            lambda gb, ph, gt: (gb * nt + jnp.where(ph == 1, gt, 0), 0)),
        out_shape=jax.ShapeDtypeStruct((bt, 1), jnp.float32),
        scratch_shapes=[pltpu.VMEM((NE, c), jnp.float32)],
    )(p, d3, c3, bsum)

    return out.reshape(b, t)
